# baseline (speedup 1.0000x reference)
import jax
import jax.numpy as jnp
from jax import lax
from jax.experimental import pallas as pl
from jax.experimental.pallas import tpu as pltpu
from jax.experimental.pallas import tpu_sc as plsc

B = 128
K = 8
LOCAL_E = 8
H = 2048
HQ = H // 4
I = 1024
IT = 512
NI = I // IT

_L = 16
_NCHUNK = B // _L


def _dispatch_body(ids_hbm, scl_hbm, act_hbm, w_hbm, ids_v, scl_v, act_v, w_v):
    wid = lax.axis_index("s") * 2 + lax.axis_index("c")

    @pl.when(wid < _NCHUNK)
    def _():
        base = wid * _L
        pltpu.sync_copy(ids_hbm.at[pl.ds(wid * K * _L, K * _L)], ids_v)
        pltpu.sync_copy(scl_hbm.at[pl.ds(wid * K * _L, K * _L)], scl_v)
        pltpu.sync_copy(act_hbm.at[pl.ds(base, _L)], act_v)
        actv = act_v[...]
        for e in range(LOCAL_E):
            acc = jnp.zeros((_L,), jnp.float32)
            for k in range(K):
                acc = acc + jnp.where(ids_v[pl.ds(k * _L, _L)] == e,
                                      scl_v[pl.ds(k * _L, _L)], 0.0)
            w_v[...] = acc * actv
            pltpu.sync_copy(w_v, w_hbm.at[pl.ds(e * B + base, _L)])


def _sc_dispatch(ids_c, scl_c, act_f):
    mesh = plsc.VectorSubcoreMesh(core_axis_name="c", subcore_axis_name="s")
    return pl.kernel(
        _dispatch_body,
        mesh=mesh,
        out_type=jax.ShapeDtypeStruct((LOCAL_E * B,), jnp.float32),
        scratch_types=[
            pltpu.VMEM((K * _L,), jnp.int32),
            pltpu.VMEM((K * _L,), jnp.float32),
            pltpu.VMEM((_L,), jnp.float32),
            pltpu.VMEM((_L,), jnp.float32),
        ],
    )(ids_c, scl_c, act_f)


def _ffn_body(w_ref, x_ref, smooth_ref,
              w1g0_ref, w1g1_ref, w1g2_ref, w1g3_ref,
              w1u0_ref, w1u1_ref, w1u2_ref, w1u3_ref,
              s1g_ref, s1u_ref,
              w20_ref, w21_ref, w22_ref, w23_ref, s2_ref,
              out_ref):
    e = pl.program_id(0)
    i = pl.program_id(1)

    w_col = w_ref[0]
    xs = x_ref[...] * smooth_ref[0]
    w1g = (w1g0_ref, w1g1_ref, w1g2_ref, w1g3_ref)
    w1u = (w1u0_ref, w1u1_ref, w1u2_ref, w1u3_ref)
    g = s1g_ref[0] * sum(
        jnp.dot(xs[:, q * HQ:(q + 1) * HQ], w1g[q][0],
                preferred_element_type=jnp.float32) for q in range(4))
    u = s1u_ref[0] * sum(
        jnp.dot(xs[:, q * HQ:(q + 1) * HQ], w1u[q][0],
                preferred_element_type=jnp.float32) for q in range(4))
    a = (g * jax.nn.sigmoid(g)) * u

    @pl.when((e == 0) & (i == 0))
    def _():
        out_ref[...] = jnp.zeros_like(out_ref)

    s2 = s2_ref[0]
    w2 = (w20_ref, w21_ref, w22_ref, w23_ref)
    for q in range(4):
        part = jnp.dot(a, w2[q][0], preferred_element_type=jnp.float32)
        sl = slice(q * HQ, (q + 1) * HQ)
        out_ref[:, sl] += part * s2[:, sl] * w_col


def kernel(x, expert_ids, smooth_scales, expert_scales, x_active_mask,
           gmm1_weight, gmm1_weight_scale, gmm2_weight, gmm2_weight_scale):
    ids_c = expert_ids.reshape(_NCHUNK, _L, K).transpose(0, 2, 1).reshape(-1)
    scl_c = expert_scales.reshape(_NCHUNK, _L, K).transpose(0, 2, 1).reshape(-1)
    act_f = x_active_mask.astype(jnp.float32)
    w_tab = _sc_dispatch(ids_c, scl_c, act_f)
    w3 = w_tab.reshape(LOCAL_E, B, 1)

    smooth3 = smooth_scales.reshape(LOCAL_E, 1, H)
    s1_3 = gmm1_weight_scale.reshape(LOCAL_E, 1, 2 * I)
    s2_3 = gmm2_weight_scale.reshape(LOCAL_E, 1, H)

    grid = (LOCAL_E, NI)

    def w1g_spec(q):
        return pl.BlockSpec((1, HQ, IT), lambda e, i, q=q: (e, q, i))

    def w1u_spec(q):
        return pl.BlockSpec((1, HQ, IT), lambda e, i, q=q: (e, q, NI + i))

    def w2_spec(q):
        return pl.BlockSpec((1, IT, HQ), lambda e, i, q=q: (e, i, q))

    out = pl.pallas_call(
        _ffn_body,
        grid=grid,
        in_specs=[
            pl.BlockSpec((1, B, 1), lambda e, i: (e, 0, 0)),
            pl.BlockSpec((B, H), lambda e, i: (0, 0)),
            pl.BlockSpec((1, 1, H), lambda e, i: (e, 0, 0)),
            w1g_spec(0), w1g_spec(1), w1g_spec(2), w1g_spec(3),
            w1u_spec(0), w1u_spec(1), w1u_spec(2), w1u_spec(3),
            pl.BlockSpec((1, 1, IT), lambda e, i: (e, 0, i)),
            pl.BlockSpec((1, 1, IT), lambda e, i: (e, 0, NI + i)),
            w2_spec(0), w2_spec(1), w2_spec(2), w2_spec(3),
            pl.BlockSpec((1, 1, H), lambda e, i: (e, 0, 0)),
        ],
        out_specs=pl.BlockSpec((B, H), lambda e, i: (0, 0)),
        out_shape=jax.ShapeDtypeStruct((B, H), jnp.float32),
        compiler_params=pltpu.CompilerParams(
            dimension_semantics=("arbitrary", "arbitrary"),
        ),
    )(w3, x, smooth3,
      gmm1_weight, gmm1_weight, gmm1_weight, gmm1_weight,
      gmm1_weight, gmm1_weight, gmm1_weight, gmm1_weight,
      s1_3, s1_3,
      gmm2_weight, gmm2_weight, gmm2_weight, gmm2_weight, s2_3)
    return out


# speedup vs baseline: 1.2373x; 1.2373x over previous
import jax
import jax.numpy as jnp
from jax.experimental import pallas as pl
from jax.experimental.pallas import tpu as pltpu

B = 128
K = 8
LOCAL_E = 8
H = 2048
HQ = H // 4
I = 1024
IT = 512
NI = I // IT


def _ffn_body(ids_ref, scl_ref, act_ref, x_ref, smooth_ref,
              w1g0_ref, w1g1_ref, w1g2_ref, w1g3_ref,
              w1u0_ref, w1u1_ref, w1u2_ref, w1u3_ref,
              s1g_ref, s1u_ref,
              w20_ref, w21_ref, w22_ref, w23_ref, s2_ref,
              out_ref):
    e = pl.program_id(0)
    i = pl.program_id(1)

    m = (ids_ref[...] == e).astype(jnp.float32)
    w_col = jnp.sum(m * scl_ref[...], axis=1, keepdims=True) * act_ref[...]

    xs = x_ref[...] * smooth_ref[0]
    w1g = (w1g0_ref, w1g1_ref, w1g2_ref, w1g3_ref)
    w1u = (w1u0_ref, w1u1_ref, w1u2_ref, w1u3_ref)
    g = s1g_ref[0] * sum(
        jnp.dot(xs[:, q * HQ:(q + 1) * HQ], w1g[q][0],
                preferred_element_type=jnp.float32) for q in range(4))
    u = s1u_ref[0] * sum(
        jnp.dot(xs[:, q * HQ:(q + 1) * HQ], w1u[q][0],
                preferred_element_type=jnp.float32) for q in range(4))
    a = (g * jax.nn.sigmoid(g)) * u

    @pl.when((e == 0) & (i == 0))
    def _():
        out_ref[...] = jnp.zeros_like(out_ref)

    s2 = s2_ref[0]
    w2 = (w20_ref, w21_ref, w22_ref, w23_ref)
    for q in range(4):
        part = jnp.dot(a, w2[q][0], preferred_element_type=jnp.float32)
        sl = slice(q * HQ, (q + 1) * HQ)
        out_ref[:, sl] += part * s2[:, sl] * w_col


def kernel(x, expert_ids, smooth_scales, expert_scales, x_active_mask,
           gmm1_weight, gmm1_weight_scale, gmm2_weight, gmm2_weight_scale):
    act_col = x_active_mask.astype(jnp.float32).reshape(B, 1)
    smooth3 = smooth_scales.reshape(LOCAL_E, 1, H)
    s1_3 = gmm1_weight_scale.reshape(LOCAL_E, 1, 2 * I)
    s2_3 = gmm2_weight_scale.reshape(LOCAL_E, 1, H)

    grid = (LOCAL_E, NI)

    def w1g_spec(q):
        return pl.BlockSpec((1, HQ, IT), lambda e, i, q=q: (e, q, i))

    def w1u_spec(q):
        return pl.BlockSpec((1, HQ, IT), lambda e, i, q=q: (e, q, NI + i))

    def w2_spec(q):
        return pl.BlockSpec((1, IT, HQ), lambda e, i, q=q: (e, i, q))

    out = pl.pallas_call(
        _ffn_body,
        grid=grid,
        in_specs=[
            pl.BlockSpec((B, K), lambda e, i: (0, 0)),
            pl.BlockSpec((B, K), lambda e, i: (0, 0)),
            pl.BlockSpec((B, 1), lambda e, i: (0, 0)),
            pl.BlockSpec((B, H), lambda e, i: (0, 0)),
            pl.BlockSpec((1, 1, H), lambda e, i: (e, 0, 0)),
            w1g_spec(0), w1g_spec(1), w1g_spec(2), w1g_spec(3),
            w1u_spec(0), w1u_spec(1), w1u_spec(2), w1u_spec(3),
            pl.BlockSpec((1, 1, IT), lambda e, i: (e, 0, i)),
            pl.BlockSpec((1, 1, IT), lambda e, i: (e, 0, NI + i)),
            w2_spec(0), w2_spec(1), w2_spec(2), w2_spec(3),
            pl.BlockSpec((1, 1, H), lambda e, i: (e, 0, 0)),
        ],
        out_specs=pl.BlockSpec((B, H), lambda e, i: (0, 0)),
        out_shape=jax.ShapeDtypeStruct((B, H), jnp.float32),
        compiler_params=pltpu.CompilerParams(
            dimension_semantics=("arbitrary", "arbitrary"),
        ),
    )(expert_ids, expert_scales, act_col, x, smooth3,
      gmm1_weight, gmm1_weight, gmm1_weight, gmm1_weight,
      gmm1_weight, gmm1_weight, gmm1_weight, gmm1_weight,
      s1_3, s1_3,
      gmm2_weight, gmm2_weight, gmm2_weight, gmm2_weight, s2_3)
    return out
